# baseline (speedup 1.0000x reference)
import functools

import jax
import jax.numpy as jnp
from jax import lax
from jax.experimental import pallas as pl
from jax.experimental.pallas import tpu as pltpu
from jax.experimental.pallas import tpu_sc as plsc

VOCAB = 1_000_000
D = 20
PAD_IDX = 4
PITCH = 32

_info = plsc.get_sparse_core_info()
NC, NS, L = _info.num_cores, _info.num_subcores, _info.num_lanes
NW = NC * NS

_CP = pltpu.CompilerParams(use_tc_tiling_on_sc=False, needs_layout_passes=False)

W1 = 800
NBLK = VOCAB // W1
V1 = W1 // L


def _relayout_body(tbl_hbm, scr_hbm, in_v, out_v):
    wid = lax.axis_index("s") * NC + lax.axis_index("c")
    lane = lax.iota(jnp.int32, L)

    def block(c, _):
        b = wid + NW * c

        @pl.when(b < NBLK)
        def _():
            c0 = b * W1
            for d in range(D):
                pltpu.sync_copy(tbl_hbm.at[d, pl.ds(c0, W1)], in_v.at[d])

            def shuffle(v, _):
                rowv = v * L + lane
                for d in range(D):
                    plsc.store_scatter(
                        out_v, [rowv, jnp.full((L,), d, jnp.int32)],
                        in_v[d, pl.ds(v * L, L)],
                    )
                return _
            lax.fori_loop(0, V1, shuffle, None, unroll=False)
            pltpu.sync_copy(out_v, scr_hbm.at[pl.ds(c0, W1)])
        return _

    lax.fori_loop(0, (NBLK + NW - 1) // NW, block, None, unroll=False)


B_TOTAL = 16384 * 30
GROUPS = B_TOTAL // 128
G_PER_W = GROUPS // NW
GQ = 8
N_CHUNKS = G_PER_W // GQ


def _gather_body(x_hbm, scr_hbm, out_hbm, idx_v, pad_v, blk_v, sem):
    wid = lax.axis_index("s") * NC + lax.axis_index("c")
    lane = lax.iota(jnp.int32, L)
    zeros = jnp.zeros((L,), jnp.float32)
    ones = jnp.ones((L,), jnp.float32)

    def chunk(g, _):
        r0 = wid * G_PER_W + g * GQ
        pltpu.sync_copy(x_hbm.at[pl.ds(r0, GQ)], idx_v)

        copies = [
            pltpu.async_copy(
                scr_hbm.at[idx_v.at[q]],
                pad_v.at[pl.ds(q * 128, 128)],
                sem,
            )
            for q in range(GQ)
        ]
        for cp in copies:
            cp.wait()

        for q in range(GQ):
            r = r0 + q
            j = r // 128
            s0 = (r % 128) * 128

            def extract(u, _):
                xr = idx_v[q, pl.ds(u * L, L)]
                mf = jnp.where(xr == PAD_IDX, zeros, ones)
                rowv = q * 128 + u * L + lane
                for d in range(D):
                    val = plsc.load_gather(
                        pad_v, [rowv, jnp.full((L,), d, jnp.int32)]
                    ) * mf
                    blk_v[d, pl.ds(u * L, L)] = val
                return _
            lax.fori_loop(0, 128 // L, extract, None, unroll=False)

            for d in range(D):
                pltpu.sync_copy(blk_v.at[d], out_hbm.at[d, j, pl.ds(s0, 128)])
        return _

    lax.fori_loop(0, N_CHUNKS, chunk, None, unroll=False)


@functools.partial(jax.jit, static_argnames=())
def kernel(x, table):
    mesh = plsc.VectorSubcoreMesh(core_axis_name="c", subcore_axis_name="s")
    tbl_t = table.T
    x2t = x.T.reshape(GROUPS, 128)

    scratch = pl.kernel(
        _relayout_body,
        out_type=jax.ShapeDtypeStruct((VOCAB, PITCH), jnp.float32),
        mesh=mesh,
        scratch_types=[
            pltpu.VMEM((D, W1), jnp.float32),
            pltpu.VMEM((W1, PITCH), jnp.float32),
        ],
        compiler_params=_CP,
    )(tbl_t)

    out_t = pl.kernel(
        _gather_body,
        out_type=jax.ShapeDtypeStruct((D, 30, 16384), jnp.float32),
        mesh=mesh,
        scratch_types=[
            pltpu.VMEM((GQ, 128), jnp.int32),
            pltpu.VMEM((GQ * 128, PITCH), jnp.float32),
            pltpu.VMEM((D, 128), jnp.float32),
            pltpu.SemaphoreType.DMA,
        ],
        compiler_params=_CP,
    )(x2t, scratch)

    return out_t.transpose(2, 1, 0)


# speedup vs baseline: 1.6539x; 1.6539x over previous
import functools

import jax
import jax.numpy as jnp
from jax import lax
from jax.experimental import pallas as pl
from jax.experimental.pallas import tpu as pltpu
from jax.experimental.pallas import tpu_sc as plsc

VOCAB = 1_000_000
D = 20
PAD_IDX = 4

_info = plsc.get_sparse_core_info()
NC, NS, L = _info.num_cores, _info.num_subcores, _info.num_lanes
NW = NC * NS

B_TOTAL = 16384 * 30
B_PER_W = B_TOTAL // NW
C = 1024
N_CHUNKS = B_PER_W // C
IDX_ROWS = C // 128
G_STREAMS = 2 * C // 128
X_ROWS_PER_W = B_PER_W // 128


def _body(x_hbm, tbl16_hbm, out_hbm, idx_v, dma_idx_v, pad_v, outb_v, sem):
    wid = lax.axis_index("s") * NC + lax.axis_index("c")
    lane = lax.iota(jnp.int32, L)
    lane2 = lane * 2
    zeros = jnp.zeros((L,), jnp.float32)
    ones = jnp.ones((L,), jnp.float32)

    def chunk(g, _):
        xrow0 = wid * X_ROWS_PER_W + g * IDX_ROWS
        base = wid * B_PER_W + g * C

        pltpu.sync_copy(x_hbm.at[pl.ds(xrow0, IDX_ROWS)], idx_v)

        def build(k, _):
            xr = idx_v[k // 8, pl.ds((k % 8) * L, L)]
            j0 = (xr * 5) >> 2
            row = jnp.full((L,), 0, jnp.int32) + (k >> 2)
            col0 = (k % 4) * 32 + lane2
            plsc.store_scatter(dma_idx_v, [row, col0], j0)
            plsc.store_scatter(dma_idx_v, [row, col0 + 1], j0 + 1)
            return _
        lax.fori_loop(0, C // L, build, None, unroll=False)

        copies = [
            pltpu.async_copy(
                tbl16_hbm.at[dma_idx_v.at[j]],
                pad_v.at[pl.ds(j * 128, 128)],
                sem,
            )
            for j in range(G_STREAMS)
        ]
        for cp in copies:
            cp.wait()

        def compact(k, _):
            xr = idx_v[k // 8, pl.ds((k % 8) * L, L)]
            off = (xr & 3) << 2
            mf = jnp.where(xr == PAD_IDX, zeros, ones)
            r = k * L + lane
            r2 = r * 2
            for c in range(D):
                t = off + c
                b = t >> 4
                row = r2 + b
                col = t - (b << 4)
                val = plsc.load_gather(pad_v, [row, col]) * mf
                plsc.store_scatter(outb_v, [r, jnp.full((L,), c, jnp.int32)], val)
            return _
        lax.fori_loop(0, C // L, compact, None, unroll=False)

        pltpu.sync_copy(outb_v, out_hbm.at[pl.ds(base, C)])
        return _

    lax.fori_loop(0, N_CHUNKS, chunk, None, unroll=False)


@functools.partial(jax.jit, static_argnames=())
def kernel(x, table):
    x2 = x.reshape(-1, 128)
    tbl16 = table.reshape(VOCAB * D // 16, 16)
    mesh = plsc.VectorSubcoreMesh(core_axis_name="c", subcore_axis_name="s")
    out = pl.kernel(
        _body,
        out_type=jax.ShapeDtypeStruct((B_TOTAL, D), jnp.float32),
        mesh=mesh,
        scratch_types=[
            pltpu.VMEM((IDX_ROWS, 128), jnp.int32),
            pltpu.VMEM((G_STREAMS, 128), jnp.int32),
            pltpu.VMEM((2 * C, 16), jnp.float32),
            pltpu.VMEM((C, D), jnp.float32),
            pltpu.SemaphoreType.DMA,
        ],
        compiler_params=pltpu.CompilerParams(
            use_tc_tiling_on_sc=False, needs_layout_passes=False
        ),
    )(x2, tbl16)
    return out.reshape(16384, 30, D)
